# baseline (speedup 1.0000x reference)
import functools

import jax
import jax.numpy as jnp
from jax import lax
from jax.experimental import pallas as pl
from jax.experimental.pallas import tpu as pltpu
from jax.experimental.pallas import tpu_sc as plsc

_ND = 13
_NF = 26
_EMB = 16
_ROW = _ND + _NF * _EMB

_NC = 2
_NS = 16
_FPC = _NF // _NC

_BB = 1024
_NG = _BB // 128
_VP = _BB + 1

_VC = 1000
_VCP = _VC + 1


def _sc_embed(n_batch, seq, f_dim, vocab):
    n_q = n_batch // _BB
    emb_units = seq * _FPC * n_q
    dense_units = seq * n_q
    emb_base, emb_extra = divmod(emb_units, _NS)
    dense_base, dense_extra = divmod(dense_units // _NC, _NS)
    conv_units = _FPC * (vocab // _VC)
    conv_base, conv_extra = divmod(conv_units, _NS)
    mesh = plsc.VectorSubcoreMesh(core_axis_name="c", subcore_axis_name="s")

    @functools.partial(
        pl.kernel,
        mesh=mesh,
        out_type=[
            jax.ShapeDtypeStruct((seq, _ROW, n_batch), jnp.float32),
            jax.ShapeDtypeStruct((_NF, vocab, _EMB), jnp.float32),
        ],
        scratch_types=[
            pltpu.VMEM((2, _BB), jnp.float32),
            pltpu.VMEM((2 * _NG, 128), jnp.int32),
            pltpu.VMEM((2 * _BB, _EMB), jnp.float32),
            pltpu.VMEM((2 * _EMB, _VP), jnp.float32),
            pltpu.VMEM((_ND, _BB), jnp.float32),
            pltpu.VMEM((_EMB, _VCP), jnp.float32),
            pltpu.VMEM((_VC, _EMB), jnp.float32),
            pltpu.SemaphoreType.DMA,
            pltpu.SemaphoreType.DMA,
            pltpu.SemaphoreType.DMA,
            pltpu.SemaphoreType.DMA,
            pltpu.SemaphoreType.DMA,
        ],
        compiler_params=pltpu.CompilerParams(
            use_tc_tiling_on_sc=False, needs_layout_passes=False),
    )
    def k(xt, tblt, out, ltbl, xi_v, idx_v, emb_v, v_out, dense_v,
          cin_v, cout_v, s0, s1, sx, sw0, sw1):
        sc = lax.axis_index("c")
        tid = lax.axis_index("s")
        lanes = lax.iota(jnp.int32, 16)

        my_conv = conv_base + jnp.where(tid < conv_extra, 1, 0)

        def conv_loop(kk, carry):
            q = jnp.minimum(kk, my_conv - 1) * _NS + tid
            fl = q // (vocab // _VC)
            f = sc * _FPC + fl
            v0 = (q % (vocab // _VC)) * _VC
            pltpu.sync_copy(
                tblt.at[f, :, pl.ds(v0, _VC)],
                cin_v.at[:, pl.ds(0, _VC)])

            def trans(vb, carry2):
                base = jnp.full((16,), vb * 16, dtype=jnp.int32)
                for i in range(16):
                    cout_v[vb * 16 + i, :] = plsc.load_gather(
                        cin_v, [lanes, base + i])
                return carry2

            lax.fori_loop(0, _VC // 16, trans, 0)
            for i in range(_VC - (_VC // 16) * 16):
                vv = (_VC // 16) * 16 + i
                cout_v[vv, :] = plsc.load_gather(
                    cin_v, [lanes, jnp.full((16,), vv, dtype=jnp.int32)])
            pltpu.sync_copy(cout_v, ltbl.at[f, pl.ds(v0, _VC)])
            return carry

        lax.fori_loop(0, conv_base + 1, conv_loop, 0)
        plsc.subcore_barrier()

        my_units = emb_base + jnp.where(tid < emb_extra, 1, 0)
        my_dense = dense_base + jnp.where(tid < dense_extra, 1, 0)

        def unit_coords(kk):
            q = jnp.minimum(kk, my_units - 1) * _NS + tid
            s = q // (_FPC * n_q)
            r = q % (_FPC * n_q)
            f = sc * _FPC + r // n_q
            b0 = (r % n_q) * _BB
            return s, f, b0

        def fetch_xi(kk, slot):
            s, f, b0 = unit_coords(kk)
            return pltpu.async_copy(
                xt.at[s, _ND + f, pl.ds(b0, _BB)],
                xi_v.at[slot], sx)

        def fire(kk, slot, sem):
            s, f, b0 = unit_coords(kk)
            for j in range(_NG):
                for p in range(8):
                    idx_v[slot * _NG + j, pl.ds(p * 16, 16)] = (
                        lax.convert_element_type(
                            xi_v[slot, pl.ds(j * 128 + p * 16, 16)],
                            jnp.int32))
            return [
                pltpu.async_copy(
                    ltbl.at[f].at[idx_v.at[slot * _NG + j]],
                    emb_v.at[pl.ds(slot * _BB + j * 128, 128)], sem)
                for j in range(_NG)
            ]

        def drain_write(slot, sem):
            pltpu.make_async_copy(
                v_out.at[pl.ds(slot * _EMB, _EMB), pl.ds(0, _BB)],
                out.at[0, pl.ds(_ND, _EMB), pl.ds(0, _BB)], sem).wait()

        def weave_write(kk, slot, sem):
            s, f, b0 = unit_coords(kk)

            def weave(bb, carry2):
                base = jnp.full((16,), bb * 16, dtype=jnp.int32)
                for i in range(16):
                    plsc.store_scatter(
                        v_out, [lanes + slot * _EMB, base + i],
                        emb_v[slot * _BB + bb * 16 + i])
                return carry2

            lax.fori_loop(0, _BB // 16, weave, 0)
            pltpu.async_copy(
                v_out.at[pl.ds(slot * _EMB, _EMB), pl.ds(0, _BB)],
                out.at[s, pl.ds(_ND + f * _EMB, _EMB), pl.ds(b0, _BB)], sem)

        def dense_loop(kk, carry):
            q = (jnp.minimum(kk, my_dense - 1) * _NS + tid) * _NC + sc
            s = q // n_q
            b0 = (q % n_q) * _BB
            pltpu.sync_copy(xt.at[s, pl.ds(0, _ND), pl.ds(b0, _BB)], dense_v)
            pltpu.sync_copy(
                dense_v, out.at[s, pl.ds(0, _ND), pl.ds(b0, _BB)])
            return carry

        lax.fori_loop(0, dense_base + 1, dense_loop, 0)

        iters = emb_base + 1
        if iters % 2:
            iters += 1

        def drain_xi():
            pltpu.make_async_copy(
                xt.at[0, 0, pl.ds(0, _BB)], xi_v.at[0], sx).wait()

        def drain_gathers(sem):
            for _ in range(_NG):
                pltpu.make_async_copy(
                    ltbl.at[0].at[idx_v.at[0]],
                    emb_v.at[pl.ds(0, 128)], sem).wait()

        fetch_xi(0, 0).wait()
        fire(0, 0, s0)
        fetch_xi(1, 1)

        def pair(m, carry):
            for par in (0, 1):
                kk = m * 2 + par
                sem, nsem = (s0, s1) if par == 0 else (s1, s0)
                semw = sw0 if par == 0 else sw1

                @pl.when(kk + 1 < iters)
                def _():
                    drain_xi()
                    fire(kk + 1, 1 - par, nsem)

                @pl.when(kk + 2 < iters)
                def _():
                    fetch_xi(kk + 2, par)

                drain_gathers(sem)

                @pl.when(kk >= 2)
                def _():
                    drain_write(par, semw)

                weave_write(kk, par, semw)
            return carry

        lax.fori_loop(0, iters // 2, pair, 0)
        drain_write(0, sw0)
        drain_write(1, sw1)

    return k


def kernel(x, tables):
    b, seq, f_dim = x.shape
    nf, vocab, emb = tables.shape
    xt = x.transpose(1, 2, 0)
    tblt = tables.transpose(0, 2, 1)
    out_t, _ = _sc_embed(b, seq, f_dim, vocab)(xt, tblt)
    return out_t.transpose(2, 0, 1)


# speedup vs baseline: 1.0612x; 1.0612x over previous
import functools

import jax
import jax.numpy as jnp
from jax import lax
from jax.experimental import pallas as pl
from jax.experimental.pallas import tpu as pltpu
from jax.experimental.pallas import tpu_sc as plsc

_ND = 13
_NF = 26
_EMB = 16
_ROW = _ND + _NF * _EMB

_NC = 2
_NS = 16
_FPC = _NF // _NC

_BB = 1024
_NG = _BB // 128
_VP = _BB + 1

_VC = 1000
_VCP = _VC + 1


def _sc_embed(n_batch, seq, f_dim, vocab):
    n_q = n_batch // _BB
    emb_units = seq * _FPC * n_q
    dense_units = seq * n_q
    emb_base, emb_extra = divmod(emb_units, _NS)
    dense_base, dense_extra = divmod(dense_units // _NC, _NS)
    conv_units = _FPC * (vocab // _VC)
    conv_base, conv_extra = divmod(conv_units, _NS)
    mesh = plsc.VectorSubcoreMesh(core_axis_name="c", subcore_axis_name="s")

    @functools.partial(
        pl.kernel,
        mesh=mesh,
        out_type=[
            jax.ShapeDtypeStruct((seq, _ROW, n_batch), jnp.float32),
            jax.ShapeDtypeStruct((_NF, vocab, _EMB), jnp.float32),
        ],
        scratch_types=[
            pltpu.VMEM((2, _BB), jnp.float32),
            pltpu.VMEM((2 * _NG, 128), jnp.int32),
            pltpu.VMEM((2 * _BB, _EMB), jnp.float32),
            pltpu.VMEM((2 * _EMB, _VP), jnp.float32),
            pltpu.VMEM((_ND, _BB), jnp.float32),
            pltpu.VMEM((2 * _EMB, _VCP), jnp.float32),
            pltpu.SemaphoreType.DMA,
            pltpu.SemaphoreType.DMA,
            pltpu.SemaphoreType.DMA,
            pltpu.SemaphoreType.DMA,
            pltpu.SemaphoreType.DMA,
        ],
        compiler_params=pltpu.CompilerParams(
            use_tc_tiling_on_sc=False, needs_layout_passes=False),
    )
    def k(xt, tblt, out, ltbl, xi_v, idx_v, emb_v, v_out, dense_v,
          cin_v, s0, s1, sx, sw0, sw1):
        sc = lax.axis_index("c")
        tid = lax.axis_index("s")
        lanes = lax.iota(jnp.int32, 16)

        my_conv = conv_base + jnp.where(tid < conv_extra, 1, 0)
        conv_iters = conv_base + 1
        if conv_iters % 2:
            conv_iters += 1

        def conv_coords(kk):
            q = jnp.minimum(kk, my_conv - 1) * _NS + tid
            f = sc * _FPC + q // (vocab // _VC)
            v0 = (q % (vocab // _VC)) * _VC
            return f, v0

        def fetch_cin(kk, slot, sem):
            f, v0 = conv_coords(kk)
            return pltpu.async_copy(
                tblt.at[f, :, pl.ds(v0, _VC)],
                cin_v.at[pl.ds(slot * _EMB, _EMB), pl.ds(0, _VC)], sem)

        def conv_unit(kk, slot):
            f, v0 = conv_coords(kk)
            rbase = lanes + slot * _EMB

            def trans(vb, carry2):
                base = jnp.full((16,), vb * 16, dtype=jnp.int32)
                for i in range(16):
                    emb_v[vb * 16 + i, :] = plsc.load_gather(
                        cin_v, [rbase, base + i])
                return carry2

            lax.fori_loop(0, _VC // 16, trans, 0)
            for i in range(_VC - (_VC // 16) * 16):
                vv = (_VC // 16) * 16 + i
                emb_v[vv, :] = plsc.load_gather(
                    cin_v, [rbase, jnp.full((16,), vv, dtype=jnp.int32)])
            pltpu.sync_copy(
                emb_v.at[pl.ds(0, _VC)], ltbl.at[f, pl.ds(v0, _VC)])

        def drain_cin(sem):
            pltpu.make_async_copy(
                tblt.at[0, :, pl.ds(0, _VC)],
                cin_v.at[pl.ds(0, _EMB), pl.ds(0, _VC)], sem).wait()

        fetch_cin(0, 0, s0)

        def conv_pair(m, carry):
            for par in (0, 1):
                kk = m * 2 + par
                sem, nsem = (s0, s1) if par == 0 else (s1, s0)

                @pl.when(kk + 1 < conv_iters)
                def _():
                    fetch_cin(kk + 1, 1 - par, nsem)

                drain_cin(sem)
                conv_unit(kk, par)
            return carry

        lax.fori_loop(0, conv_iters // 2, conv_pair, 0)
        plsc.subcore_barrier()

        my_units = emb_base + jnp.where(tid < emb_extra, 1, 0)
        my_dense = dense_base + jnp.where(tid < dense_extra, 1, 0)

        def unit_coords(kk):
            q = jnp.minimum(kk, my_units - 1) * _NS + tid
            s = q // (_FPC * n_q)
            r = q % (_FPC * n_q)
            f = sc * _FPC + r // n_q
            b0 = (r % n_q) * _BB
            return s, f, b0

        def fetch_xi(kk, slot):
            s, f, b0 = unit_coords(kk)
            return pltpu.async_copy(
                xt.at[s, _ND + f, pl.ds(b0, _BB)],
                xi_v.at[slot], sx)

        def fire(kk, slot, sem):
            s, f, b0 = unit_coords(kk)
            for j in range(_NG):
                for p in range(8):
                    idx_v[slot * _NG + j, pl.ds(p * 16, 16)] = (
                        lax.convert_element_type(
                            xi_v[slot, pl.ds(j * 128 + p * 16, 16)],
                            jnp.int32))
            return [
                pltpu.async_copy(
                    ltbl.at[f].at[idx_v.at[slot * _NG + j]],
                    emb_v.at[pl.ds(slot * _BB + j * 128, 128)], sem)
                for j in range(_NG)
            ]

        def drain_write(slot, sem):
            pltpu.make_async_copy(
                v_out.at[pl.ds(slot * _EMB, _EMB), pl.ds(0, _BB)],
                out.at[0, pl.ds(_ND, _EMB), pl.ds(0, _BB)], sem).wait()

        def weave_write(kk, slot, sem):
            s, f, b0 = unit_coords(kk)

            def weave(bb, carry2):
                base = jnp.full((16,), bb * 16, dtype=jnp.int32)
                for i in range(16):
                    plsc.store_scatter(
                        v_out, [lanes + slot * _EMB, base + i],
                        emb_v[slot * _BB + bb * 16 + i])
                return carry2

            lax.fori_loop(0, _BB // 16, weave, 0)
            pltpu.async_copy(
                v_out.at[pl.ds(slot * _EMB, _EMB), pl.ds(0, _BB)],
                out.at[s, pl.ds(_ND + f * _EMB, _EMB), pl.ds(b0, _BB)], sem)

        def dense_loop(kk, carry):
            q = (jnp.minimum(kk, my_dense - 1) * _NS + tid) * _NC + sc
            s = q // n_q
            b0 = (q % n_q) * _BB
            pltpu.sync_copy(xt.at[s, pl.ds(0, _ND), pl.ds(b0, _BB)], dense_v)
            pltpu.sync_copy(
                dense_v, out.at[s, pl.ds(0, _ND), pl.ds(b0, _BB)])
            return carry

        lax.fori_loop(0, dense_base + 1, dense_loop, 0)

        iters = emb_base + 1
        if iters % 2:
            iters += 1

        def drain_xi():
            pltpu.make_async_copy(
                xt.at[0, 0, pl.ds(0, _BB)], xi_v.at[0], sx).wait()

        def drain_gathers(sem):
            for _ in range(_NG):
                pltpu.make_async_copy(
                    ltbl.at[0].at[idx_v.at[0]],
                    emb_v.at[pl.ds(0, 128)], sem).wait()

        fetch_xi(0, 0).wait()
        fire(0, 0, s0)
        fetch_xi(1, 1)

        def pair(m, carry):
            for par in (0, 1):
                kk = m * 2 + par
                sem, nsem = (s0, s1) if par == 0 else (s1, s0)
                semw = sw0 if par == 0 else sw1

                @pl.when(kk + 1 < iters)
                def _():
                    drain_xi()
                    fire(kk + 1, 1 - par, nsem)

                @pl.when(kk + 2 < iters)
                def _():
                    fetch_xi(kk + 2, par)

                drain_gathers(sem)

                @pl.when(kk >= 2)
                def _():
                    drain_write(par, semw)

                weave_write(kk, par, semw)
            return carry

        lax.fori_loop(0, iters // 2, pair, 0)
        drain_write(0, sw0)
        drain_write(1, sw1)

    return k


def kernel(x, tables):
    b, seq, f_dim = x.shape
    nf, vocab, emb = tables.shape
    xt = x.transpose(1, 2, 0)
    tblt = tables.transpose(0, 2, 1)
    out_t, _ = _sc_embed(b, seq, f_dim, vocab)(xt, tblt)
    return out_t.transpose(2, 0, 1)


# speedup vs baseline: 1.0903x; 1.0274x over previous
import functools

import jax
import jax.numpy as jnp
from jax import lax
from jax.experimental import pallas as pl
from jax.experimental.pallas import tpu as pltpu
from jax.experimental.pallas import tpu_sc as plsc

_ND = 13
_NF = 26
_EMB = 16
_ROW = _ND + _NF * _EMB

_NC = 2
_NS = 16
_FPC = _NF // _NC

_BB = 1024
_NG = _BB // 128
_VP = _BB + 1

_VC = 1000
_VCP = _VC + 1


def _sc_embed(n_batch, seq, f_dim, vocab):
    n_q = n_batch // _BB
    emb_units = seq * _FPC * n_q
    dense_units = seq * n_q
    emb_base, emb_extra = divmod(emb_units, _NS)
    dense_base, dense_extra = divmod(dense_units // _NC, _NS)
    conv_units = _FPC * (vocab // _VC)
    conv_base, conv_extra = divmod(conv_units, _NS)
    mesh = plsc.VectorSubcoreMesh(core_axis_name="c", subcore_axis_name="s")

    @functools.partial(
        pl.kernel,
        mesh=mesh,
        out_type=[
            jax.ShapeDtypeStruct((seq, _ROW, n_batch), jnp.float32),
            jax.ShapeDtypeStruct((_NF, vocab, _EMB), jnp.float32),
        ],
        scratch_types=[
            pltpu.VMEM((2, _BB), jnp.float32),
            pltpu.VMEM((2 * _NG, 128), jnp.int32),
            pltpu.VMEM((2 * _BB, _EMB), jnp.float32),
            pltpu.VMEM((2 * _EMB, _VP), jnp.float32),
            pltpu.VMEM((_ND, _BB), jnp.float32),
            pltpu.VMEM((2 * _EMB, _VCP), jnp.float32),
            pltpu.SemaphoreType.DMA,
            pltpu.SemaphoreType.DMA,
            pltpu.SemaphoreType.DMA,
            pltpu.SemaphoreType.DMA,
            pltpu.SemaphoreType.DMA,
        ],
        compiler_params=pltpu.CompilerParams(
            use_tc_tiling_on_sc=False, needs_layout_passes=False),
    )
    def k(xt, tblt, out, ltbl, xi_v, idx_v, emb_v, v_out, dense_v,
          cin_v, s0, s1, sx, sw0, sw1):
        sc = lax.axis_index("c")
        tid = lax.axis_index("s")
        lanes = lax.iota(jnp.int32, 16)

        my_conv = conv_base + jnp.where(tid < conv_extra, 1, 0)
        conv_iters = conv_base + 1
        if conv_iters % 2:
            conv_iters += 1

        def conv_coords(kk):
            q = jnp.minimum(kk, my_conv - 1) * _NS + tid
            f = sc * _FPC + q // (vocab // _VC)
            v0 = (q % (vocab // _VC)) * _VC
            return f, v0

        def fetch_cin(kk, slot, sem):
            f, v0 = conv_coords(kk)
            return pltpu.async_copy(
                tblt.at[f, :, pl.ds(v0, _VC)],
                cin_v.at[pl.ds(slot * _EMB, _EMB), pl.ds(0, _VC)], sem)

        def conv_unit(kk, slot, semw):
            f, v0 = conv_coords(kk)
            rbase = lanes + slot * _EMB
            obase = slot * _BB

            def trans(vb, carry2):
                base = jnp.full((16,), vb * 16, dtype=jnp.int32)
                for i in range(16):
                    emb_v[obase + vb * 16 + i, :] = plsc.load_gather(
                        cin_v, [rbase, base + i])
                return carry2

            lax.fori_loop(0, _VC // 16, trans, 0)
            for i in range(_VC - (_VC // 16) * 16):
                vv = (_VC // 16) * 16 + i
                emb_v[obase + vv, :] = plsc.load_gather(
                    cin_v, [rbase, jnp.full((16,), vv, dtype=jnp.int32)])
            pltpu.async_copy(
                emb_v.at[pl.ds(obase, _VC)], ltbl.at[f, pl.ds(v0, _VC)],
                semw)

        def drain_conv_write(slot, semw):
            pltpu.make_async_copy(
                emb_v.at[pl.ds(slot * _BB, _VC)],
                ltbl.at[0, pl.ds(0, _VC)], semw).wait()

        def drain_cin(sem):
            pltpu.make_async_copy(
                tblt.at[0, :, pl.ds(0, _VC)],
                cin_v.at[pl.ds(0, _EMB), pl.ds(0, _VC)], sem).wait()

        fetch_cin(0, 0, s0)

        def conv_pair(m, carry):
            for par in (0, 1):
                kk = m * 2 + par
                sem, nsem = (s0, s1) if par == 0 else (s1, s0)
                semw = sw0 if par == 0 else sw1

                @pl.when(kk + 1 < conv_iters)
                def _():
                    fetch_cin(kk + 1, 1 - par, nsem)

                drain_cin(sem)

                @pl.when(kk >= 2)
                def _():
                    drain_conv_write(par, semw)

                conv_unit(kk, par, semw)
            return carry

        lax.fori_loop(0, conv_iters // 2, conv_pair, 0)
        drain_conv_write(0, sw0)
        drain_conv_write(1, sw1)
        plsc.subcore_barrier()

        my_units = emb_base + jnp.where(tid < emb_extra, 1, 0)
        my_dense = dense_base + jnp.where(tid < dense_extra, 1, 0)

        def unit_coords(kk):
            q = jnp.minimum(kk, my_units - 1) * _NS + tid
            s = q // (_FPC * n_q)
            r = q % (_FPC * n_q)
            f = sc * _FPC + r // n_q
            b0 = (r % n_q) * _BB
            return s, f, b0

        def fetch_xi(kk, slot):
            s, f, b0 = unit_coords(kk)
            return pltpu.async_copy(
                xt.at[s, _ND + f, pl.ds(b0, _BB)],
                xi_v.at[slot], sx)

        def fire(kk, slot, sem):
            s, f, b0 = unit_coords(kk)
            for j in range(_NG):
                for p in range(8):
                    idx_v[slot * _NG + j, pl.ds(p * 16, 16)] = (
                        lax.convert_element_type(
                            xi_v[slot, pl.ds(j * 128 + p * 16, 16)],
                            jnp.int32))
            return [
                pltpu.async_copy(
                    ltbl.at[f].at[idx_v.at[slot * _NG + j]],
                    emb_v.at[pl.ds(slot * _BB + j * 128, 128)], sem)
                for j in range(_NG)
            ]

        def drain_write(slot, sem):
            pltpu.make_async_copy(
                v_out.at[pl.ds(slot * _EMB, _EMB), pl.ds(0, _BB)],
                out.at[0, pl.ds(_ND, _EMB), pl.ds(0, _BB)], sem).wait()

        def weave_write(kk, slot, sem):
            s, f, b0 = unit_coords(kk)

            def weave(bb, carry2):
                base = jnp.full((16,), bb * 16, dtype=jnp.int32)
                for i in range(16):
                    plsc.store_scatter(
                        v_out, [lanes + slot * _EMB, base + i],
                        emb_v[slot * _BB + bb * 16 + i])
                return carry2

            lax.fori_loop(0, _BB // 16, weave, 0)
            pltpu.async_copy(
                v_out.at[pl.ds(slot * _EMB, _EMB), pl.ds(0, _BB)],
                out.at[s, pl.ds(_ND + f * _EMB, _EMB), pl.ds(b0, _BB)], sem)

        def dense_loop(kk, carry):
            q = (jnp.minimum(kk, my_dense - 1) * _NS + tid) * _NC + sc
            s = q // n_q
            b0 = (q % n_q) * _BB
            pltpu.sync_copy(xt.at[s, pl.ds(0, _ND), pl.ds(b0, _BB)], dense_v)
            pltpu.sync_copy(
                dense_v, out.at[s, pl.ds(0, _ND), pl.ds(b0, _BB)])
            return carry

        lax.fori_loop(0, dense_base + 1, dense_loop, 0)

        iters = emb_base + 1
        if iters % 2:
            iters += 1

        def drain_xi():
            pltpu.make_async_copy(
                xt.at[0, 0, pl.ds(0, _BB)], xi_v.at[0], sx).wait()

        def drain_gathers(sem):
            for _ in range(_NG):
                pltpu.make_async_copy(
                    ltbl.at[0].at[idx_v.at[0]],
                    emb_v.at[pl.ds(0, 128)], sem).wait()

        fetch_xi(0, 0).wait()
        fire(0, 0, s0)
        fetch_xi(1, 1)

        def pair(m, carry):
            for par in (0, 1):
                kk = m * 2 + par
                sem, nsem = (s0, s1) if par == 0 else (s1, s0)
                semw = sw0 if par == 0 else sw1

                @pl.when(kk + 1 < iters)
                def _():
                    drain_xi()
                    fire(kk + 1, 1 - par, nsem)

                @pl.when(kk + 2 < iters)
                def _():
                    fetch_xi(kk + 2, par)

                drain_gathers(sem)

                @pl.when(kk >= 2)
                def _():
                    drain_write(par, semw)

                weave_write(kk, par, semw)
            return carry

        lax.fori_loop(0, iters // 2, pair, 0)
        drain_write(0, sw0)
        drain_write(1, sw1)

    return k


def kernel(x, tables):
    b, seq, f_dim = x.shape
    nf, vocab, emb = tables.shape
    xt = x.transpose(1, 2, 0)
    tblt = tables.transpose(0, 2, 1)
    out_t, _ = _sc_embed(b, seq, f_dim, vocab)(xt, tblt)
    return out_t.transpose(2, 0, 1)
